# baseline (speedup 1.0000x reference)
import functools

import jax
import jax.numpy as jnp
from jax.experimental import pallas as pl
from jax.experimental.pallas import tpu as pltpu

_NEG_SLOPE = 0.2
_IN_EPS = 1e-5
_VMEM_LIMIT = 100 * 1024 * 1024
_CH = 128
_NB = 8


def _leaky(v):
    return jnp.where(v >= 0, v, _NEG_SLOPE * v)


def _fold_w3x3(w):
    Cout, Cin = w.shape[:2]
    return jnp.transpose(w, (0, 2, 3, 1)).reshape(Cout, 9 * Cin)


def _block_kernel(x_ref, reg_ref, w_ref, o_ref,
                  zbuf, slab, gbuf, *, C, H, W, base, width):
    P = H * W
    zero = jnp.zeros((), slab.dtype)
    col = jax.lax.broadcasted_iota(jnp.int32, (1, P), 1) % W
    edge_l = col != 0
    edge_r = col != (W - 1)

    for i in range(_NB):
        zbuf[:, i * width:i * width + base] = jnp.zeros((C, base), zbuf.dtype)
        zbuf[:, i * width + base + P:(i + 1) * width] = jnp.zeros(
            (C, width - base - P), zbuf.dtype)

    def build_slab():
        for dy in range(3):
            for dx in range(3):
                t = dy * 3 + dx
                off = (dy - 1) * W + (dx - 1)
                for i in range(_NB):
                    v = zbuf[:, i * width + base + off:
                             i * width + base + off + P]
                    if dx == 0:
                        v = jnp.where(edge_l, v, zero)
                    elif dx == 2:
                        v = jnp.where(edge_r, v, zero)
                    slab[t * C:(t + 1) * C, i * P:(i + 1) * P] = v

    def conv(w):
        return jax.lax.dot_general(
            w, slab[...], (((1,), (0,)), ((), ())),
            preferred_element_type=jnp.float32)

    colf = jax.lax.broadcasted_iota(jnp.int32, (1, _NB * P), 1) % W
    edge_lf = colf != 0
    edge_rf = colf != (W - 1)

    def conv_taps(r0):
        acc = None
        for dy in range(3):
            for dx in range(3):
                t = dy * 3 + dx
                off = (dy - 1) * W + (dx - 1)
                v = jnp.concatenate(
                    [zbuf[:, i * width + base + off:
                          i * width + base + off + P] for i in range(_NB)],
                    axis=1)
                if dx == 0:
                    v = jnp.where(edge_lf, v, zero)
                elif dx == 2:
                    v = jnp.where(edge_rf, v, zero)
                d = jax.lax.dot_general(
                    w_ref[r0:r0 + C, t * C:(t + 1) * C], v,
                    (((1,), (0,)), ((), ())),
                    preferred_element_type=jnp.float32)
                acc = d if acc is None else acc + d
        return acc

    inv = 1.0 / P

    def csain_to_zbuf(read_chunk, g_row):
        for i in range(_NB):
            for c0 in range(0, C, _CH):
                v = read_chunk(i, c0).astype(jnp.float32)
                s = jnp.sum(v, axis=1, keepdims=True)
                s2 = jnp.sum(v * v, axis=1, keepdims=True)
                mean = s * inv
                var = jnp.maximum(s2 * inv - mean * mean, 0.0)
                xn = (v - mean) * jax.lax.rsqrt(var + _IN_EPS)
                g = gbuf[g_row + c0:g_row + c0 + _CH,
                         i * P:(i + 1) * P].astype(jnp.float32)
                b = gbuf[g_row + C + c0:g_row + C + c0 + _CH,
                         i * P:(i + 1) * P].astype(jnp.float32)
                y = _leaky((1.0 + g) * xn + b)
                zbuf[c0:c0 + _CH, i * width + base:i * width + base + P] = (
                    y.astype(zbuf.dtype))

    for i in range(_NB):
        zbuf[:, i * width + base:i * width + base + P] = (
            reg_ref[i].astype(zbuf.dtype))
    for i in range(4):
        gbuf[i * C:(i + 1) * C, :] = _leaky(
            conv_taps(i * C)).astype(gbuf.dtype)

    csain_to_zbuf(lambda i, c0: x_ref[i, c0:c0 + _CH, :], 0)
    h1 = conv_taps(4 * C)

    csain_to_zbuf(lambda i, c0: h1[c0:c0 + _CH, i * P:(i + 1) * P], 2 * C)
    y = conv_taps(5 * C)
    for i in range(_NB):
        o_ref[i] = y[:, i * P:(i + 1) * P] + x_ref[i]


def kernel(x, reg, Wg1, Wb1, Wc1, Wg2, Wb2, Wc2):
    N, C, H, W = x.shape
    P = H * W
    x_pp = x.reshape(N, C, P)
    reg_pp = reg.reshape(N, C, P)

    w_all = _fold_w3x3(
        jnp.concatenate([Wg1, Wb1, Wg2, Wb2, Wc1, Wc2], axis=0)
    ).astype(jnp.bfloat16)
    K9 = w_all.shape[1]

    base = max(64, W + 1)
    width = -(-(2 * base + P) // 128) * 128

    body = functools.partial(_block_kernel, C=C, H=H, W=W, base=base,
                             width=width)
    out = pl.pallas_call(
        body,
        out_shape=jax.ShapeDtypeStruct((N, C, P), jnp.float32),
        grid=(N // _NB,),
        in_specs=[
            pl.BlockSpec((_NB, C, P), lambda n: (n, 0, 0)),
            pl.BlockSpec((_NB, C, P), lambda n: (n, 0, 0)),
            pl.BlockSpec((6 * C, K9), lambda n: (0, 0)),
        ],
        out_specs=pl.BlockSpec((_NB, C, P), lambda n: (n, 0, 0)),
        scratch_shapes=[
            pltpu.VMEM((C, _NB * width), jnp.bfloat16),
            pltpu.VMEM((9 * C, _NB * P), jnp.bfloat16),
            pltpu.VMEM((4 * C, _NB * P), jnp.float32),
        ],
        compiler_params=pltpu.CompilerParams(
            dimension_semantics=("arbitrary",),
            vmem_limit_bytes=_VMEM_LIMIT),
    )(x_pp, reg_pp, w_all)
    return out.reshape(N, C, H, W)


# speedup vs baseline: 1.6490x; 1.6490x over previous
import functools

import jax
import jax.numpy as jnp
from jax.experimental import pallas as pl
from jax.experimental.pallas import tpu as pltpu

_NEG_SLOPE = 0.2
_IN_EPS = 1e-5
_VMEM_LIMIT = 100 * 1024 * 1024
_CH = 128
_NB = 4


def _leaky(v):
    return jnp.where(v >= 0, v, _NEG_SLOPE * v)


def _fold_w3x3(w):
    Cout, Cin = w.shape[:2]
    return jnp.transpose(w, (0, 2, 3, 1)).reshape(Cout, 9 * Cin)


def _block_kernel(x_ref, reg_ref, w_ref, o_ref,
                  zbuf, slab, gbuf, *, C, H, W, base, width):
    P = H * W
    zero = jnp.zeros((), slab.dtype)
    col = jax.lax.broadcasted_iota(jnp.int32, (1, P), 1) % W
    edge_l = col != 0
    edge_r = col != (W - 1)

    for i in range(_NB):
        zbuf[:, i * width:i * width + base] = jnp.zeros((C, base), zbuf.dtype)
        zbuf[:, i * width + base + P:(i + 1) * width] = jnp.zeros(
            (C, width - base - P), zbuf.dtype)

    def build_slab():
        for dy in range(3):
            for dx in range(3):
                t = dy * 3 + dx
                off = (dy - 1) * W + (dx - 1)
                for i in range(_NB):
                    v = zbuf[:, i * width + base + off:
                             i * width + base + off + P]
                    if dx == 0:
                        v = jnp.where(edge_l, v, zero)
                    elif dx == 2:
                        v = jnp.where(edge_r, v, zero)
                    slab[t * C:(t + 1) * C, i * P:(i + 1) * P] = v

    def conv(w):
        return jax.lax.dot_general(
            w, slab[...], (((1,), (0,)), ((), ())),
            preferred_element_type=jnp.float32)

    colf = jax.lax.broadcasted_iota(jnp.int32, (1, _NB * P), 1) % W
    edge_lf = colf != 0
    edge_rf = colf != (W - 1)

    def conv_taps(r0):
        acc = None
        for dy in range(3):
            for dx in range(3):
                t = dy * 3 + dx
                off = (dy - 1) * W + (dx - 1)
                v = jnp.concatenate(
                    [zbuf[:, i * width + base + off:
                          i * width + base + off + P] for i in range(_NB)],
                    axis=1)
                if dx == 0:
                    v = jnp.where(edge_lf, v, zero)
                elif dx == 2:
                    v = jnp.where(edge_rf, v, zero)
                d = jax.lax.dot_general(
                    w_ref[r0:r0 + C, t * C:(t + 1) * C], v,
                    (((1,), (0,)), ((), ())),
                    preferred_element_type=jnp.float32)
                acc = d if acc is None else acc + d
        return acc

    inv = 1.0 / P

    def csain_to_zbuf(read_chunk, g_row):
        for i in range(_NB):
            for c0 in range(0, C, _CH):
                v = read_chunk(i, c0).astype(jnp.float32)
                s = jnp.sum(v, axis=1, keepdims=True)
                s2 = jnp.sum(v * v, axis=1, keepdims=True)
                mean = s * inv
                var = jnp.maximum(s2 * inv - mean * mean, 0.0)
                xn = (v - mean) * jax.lax.rsqrt(var + _IN_EPS)
                g = gbuf[g_row + c0:g_row + c0 + _CH,
                         i * P:(i + 1) * P].astype(jnp.float32)
                b = gbuf[g_row + C + c0:g_row + C + c0 + _CH,
                         i * P:(i + 1) * P].astype(jnp.float32)
                y = _leaky((1.0 + g) * xn + b)
                zbuf[c0:c0 + _CH, i * width + base:i * width + base + P] = (
                    y.astype(zbuf.dtype))

    for i in range(_NB):
        zbuf[:, i * width + base:i * width + base + P] = (
            reg_ref[i].astype(zbuf.dtype))
    build_slab()
    for i in range(4):
        gbuf[i * C:(i + 1) * C, :] = _leaky(
            conv(w_ref[i * C:(i + 1) * C, :])).astype(gbuf.dtype)

    csain_to_zbuf(lambda i, c0: x_ref[i, c0:c0 + _CH, :], 0)
    h1 = conv_taps(4 * C)

    csain_to_zbuf(lambda i, c0: h1[c0:c0 + _CH, i * P:(i + 1) * P], 2 * C)
    y = conv_taps(5 * C)
    for i in range(_NB):
        o_ref[i] = y[:, i * P:(i + 1) * P] + x_ref[i]


def kernel(x, reg, Wg1, Wb1, Wc1, Wg2, Wb2, Wc2):
    N, C, H, W = x.shape
    P = H * W
    x_pp = x.reshape(N, C, P)
    reg_pp = reg.reshape(N, C, P)

    w_all = _fold_w3x3(
        jnp.concatenate([Wg1, Wb1, Wg2, Wb2, Wc1, Wc2], axis=0)
    ).astype(jnp.bfloat16)
    K9 = w_all.shape[1]

    base = max(64, W + 1)
    width = -(-(2 * base + P) // 128) * 128

    body = functools.partial(_block_kernel, C=C, H=H, W=W, base=base,
                             width=width)
    out = pl.pallas_call(
        body,
        out_shape=jax.ShapeDtypeStruct((N, C, P), jnp.float32),
        grid=(N // _NB,),
        in_specs=[
            pl.BlockSpec((_NB, C, P), lambda n: (n, 0, 0)),
            pl.BlockSpec((_NB, C, P), lambda n: (n, 0, 0)),
            pl.BlockSpec((6 * C, K9), lambda n: (0, 0)),
        ],
        out_specs=pl.BlockSpec((_NB, C, P), lambda n: (n, 0, 0)),
        scratch_shapes=[
            pltpu.VMEM((C, _NB * width), jnp.bfloat16),
            pltpu.VMEM((9 * C, _NB * P), jnp.bfloat16),
            pltpu.VMEM((4 * C, _NB * P), jnp.float32),
        ],
        compiler_params=pltpu.CompilerParams(
            dimension_semantics=("arbitrary",),
            vmem_limit_bytes=_VMEM_LIMIT),
    )(x_pp, reg_pp, w_all)
    return out.reshape(N, C, H, W)
